# baseline (speedup 1.0000x reference)
import jax
import jax.numpy as jnp
from jax import lax
from jax.experimental import pallas as pl
from jax.experimental.pallas import tpu as pltpu
from jax.experimental.pallas import tpu_sc as plsc

_B, _N = 64, 256
_L = 16
_NCH = _N // _L
_NEG = float(jnp.finfo(jnp.float32).min)
_NUM_CORES = 2
_NUM_SUBCORES = 16
_PER_WORKER = _B // (_NUM_CORES * _NUM_SUBCORES)


def _greedy_body(inp_hbm, out_hbm, a_ref, row_max, row_arg, colneg,
                 tmpi, sem):
    lanes = lax.iota(jnp.int32, _L)
    lane0 = lanes == 0
    zeros_f = jnp.zeros((_L,), jnp.float32)
    neg_f = jnp.full((_L,), _NEG, jnp.float32)
    ones_f = jnp.ones((_L,), jnp.float32)

    wid = lax.axis_index("s") * _NUM_CORES + lax.axis_index("c")

    def rowscan(r):
        rbase = r * _N
        bv = neg_f
        bc = jnp.zeros((_L,), jnp.int32)
        for j in range(_NCH):
            col = j * _L + lanes
            av = a_ref[pl.ds(rbase + j * _L, _L)]
            upd = av > bv
            bv = jnp.where(upd, av, bv)
            bc = jnp.where(upd, col, bc)
        return bv, bc

    def finalize(bv, bc):
        m = jnp.max(bv)
        c = jnp.min(jnp.where(bv >= m, bc, _N))
        return m, c

    def rowpass(r):
        rbase = r * _N
        bv = neg_f
        bc = jnp.zeros((_L,), jnp.int32)
        for j in range(_NCH):
            col = j * _L + lanes
            av = a_ref[pl.ds(rbase + j * _L, _L)]
            v = av + colneg[pl.ds(j * _L, _L)]
            upd = v > bv
            bv = jnp.where(upd, v, bv)
            bc = jnp.where(upd, col, bc)
        return finalize(bv, bc)

    for k in range(_PER_WORKER):
        b = wid * _PER_WORKER + k

        with jax.named_scope("sc_load"):
            def fire_row(r, carry):
                pltpu.make_async_copy(
                    inp_hbm.at[b, r], a_ref.at[pl.ds(r * _N, _N)], sem
                ).start()
                return carry

            lax.fori_loop(0, _N, fire_row, 0)

            for j in range(_NCH):
                colneg[pl.ds(j * _L, _L)] = zeros_f

            def drain_row(r, carry):
                pltpu.make_async_copy(
                    inp_hbm.at[b, r], a_ref.at[pl.ds(r * _N, _N)], sem
                ).wait()
                return carry

            lax.fori_loop(0, _N, drain_row, 0)

        with jax.named_scope("sc_init"):
            def init_quad(i, carry):
                r0 = i * 4
                scans = [rowscan(r0 + d) for d in range(4)]
                fins = [finalize(bv, bc) for bv, bc in scans]
                for d, (m, c) in enumerate(fins):
                    plsc.store_scatter(
                        row_max, [jnp.full((_L,), r0 + d, jnp.int32)],
                        jnp.full((_L,), m), mask=lane0)
                    plsc.store_scatter(
                        row_arg, [jnp.full((_L,), r0 + d, jnp.int32)],
                        jnp.full((_L,), c, jnp.int32), mask=lane0)
                return carry

            lax.fori_loop(0, _N // 4, init_quad, 0)

        bv16 = neg_f
        br16 = jnp.zeros((_L,), jnp.int32)
        for j in range(_NCH):
            ch = row_max[pl.ds(j * _L, _L)]
            mj = jnp.max(ch)
            fj = jnp.zeros((_L,), jnp.int32) + plsc.all_reduce_ffs(ch >= mj)
            sel = lanes == j
            bv16 = jnp.where(sel, mj, bv16)
            br16 = jnp.where(sel, j * _L + fj, br16)
        tmpi[...] = br16

        def not_done(st):
            return st[0] < jnp.int32(_N)

        def attempt(st):
            cnt, bv16 = st
            m = jnp.max(bv16)
            fvec = (jnp.zeros((_L,), jnp.int32)
                    + plsc.all_reduce_ffs(bv16 >= m))
            rvec = plsc.load_gather(tmpi, [fvec])
            cvec = plsc.load_gather(row_arg, [rvec])
            cmask_v = plsc.load_gather(colneg, [cvec])
            r = rvec[0]
            ok = cmask_v[0] == jnp.float32(0.0)

            @pl.when(ok)
            def _assign():
                plsc.store_scatter(colneg, [cvec], neg_f, mask=lane0)
                plsc.store_scatter(row_max, [rvec], neg_f, mask=lane0)

            @pl.when(jnp.logical_not(ok))
            def _refresh():
                nm, nc = rowpass(r)
                plsc.store_scatter(
                    row_max, [rvec], jnp.full((_L,), nm), mask=lane0)
                plsc.store_scatter(
                    row_arg, [rvec], jnp.full((_L,), nc), mask=lane0)

            j = lax.shift_right_logical(r, 4)
            ch = row_max[pl.ds(j * _L, _L)]
            mj = jnp.max(ch)
            fj = (jnp.zeros((_L,), jnp.int32)
                  + plsc.all_reduce_ffs(ch >= mj))
            selj = lanes == j
            bv16 = jnp.where(selj, mj, bv16)
            plsc.store_scatter(tmpi, [jnp.full((_L,), j, jnp.int32)],
                               j * _L + fj, mask=lane0)

            return (cnt + jnp.where(ok, jnp.int32(1), jnp.int32(0)), bv16)

        with jax.named_scope("sc_greedy"):
            lax.while_loop(not_done, attempt, (jnp.int32(0), bv16))

        def zero_row(r, carry):
            rb = r * _N
            for j in range(_NCH):
                a_ref[pl.ds(rb + j * _L, _L)] = zeros_f
            return carry

        with jax.named_scope("sc_emit"):
            lax.fori_loop(0, _N, zero_row, 0)
            for j in range(_NCH):
                rows = j * _L + lanes
                cols = row_arg[pl.ds(j * _L, _L)]
                plsc.store_scatter(a_ref, [rows * _N + cols], ones_f)

            def fire_out(r, carry):
                pltpu.make_async_copy(
                    a_ref.at[pl.ds(r * _N, _N)], out_hbm.at[b, r], sem
                ).start()
                return carry

            lax.fori_loop(0, _N, fire_out, 0)

            def drain_out(r, carry):
                pltpu.make_async_copy(
                    a_ref.at[pl.ds(r * _N, _N)], out_hbm.at[b, r], sem
                ).wait()
                return carry

            lax.fori_loop(0, _N, drain_out, 0)


@jax.jit
def _greedy_hard_perm_sc(soft_perm):
    mesh = plsc.VectorSubcoreMesh(
        core_axis_name="c", subcore_axis_name="s",
        num_cores=_NUM_CORES, num_subcores=_NUM_SUBCORES)
    return pl.kernel(
        _greedy_body,
        out_type=jax.ShapeDtypeStruct((_B, _N, _N), jnp.float32),
        mesh=mesh,
        compiler_params=pltpu.CompilerParams(needs_layout_passes=False),
        scratch_types=[
            pltpu.VMEM((_N * _N,), jnp.float32),
            pltpu.VMEM((_N,), jnp.float32),
            pltpu.VMEM((_N,), jnp.int32),
            pltpu.VMEM((_N,), jnp.float32),
            pltpu.VMEM((_L,), jnp.int32),
            pltpu.SemaphoreType.DMA,
        ],
    )(soft_perm)


def kernel(soft_perm):
    return lax.stop_gradient(_greedy_hard_perm_sc(soft_perm))
